# baseline (speedup 1.0000x reference)
import functools

import jax
import jax.numpy as jnp
from jax.experimental import pallas as pl
from jax.experimental.pallas import tpu as pltpu

_D_NODE = 48
_D_SH = 1
_D_OUT = 48
_INV_SQRT_FAN = 1.0 / (48.0 ** 0.5)

_E_TILE = 256


def _edge_body(a_ref, x_ref, sh_ref, w1_ref, b1_ref, w2_ref, b2_ref, tp_ref):
    h = jnp.dot(a_ref[...], w1_ref[...], preferred_element_type=jnp.float32)
    h = jnp.maximum(h + b1_ref[...], 0.0)
    wr = jnp.dot(h, w2_ref[...], preferred_element_type=jnp.float32) + b2_ref[...]
    et = a_ref.shape[0]
    wr3 = wr.reshape(et, _D_NODE, _D_OUT)
    tp = jnp.sum(wr3 * x_ref[...][:, :, None], axis=1)
    tp_ref[...] = tp * (sh_ref[...] * _INV_SQRT_FAN)


def _edge_compute(edge_attr, x, edge_sh, W1, b1, W2, b2, interpret=False):
    E, d_edge = edge_attr.shape
    w_numel = W2.shape[1]
    grid = (E // _E_TILE,)
    return pl.pallas_call(
        _edge_body,
        grid=grid,
        in_specs=[
            pl.BlockSpec((_E_TILE, d_edge), lambda i: (i, 0)),
            pl.BlockSpec((_E_TILE, _D_NODE), lambda i: (i, 0)),
            pl.BlockSpec((_E_TILE, _D_SH), lambda i: (i, 0)),
            pl.BlockSpec((d_edge, d_edge), lambda i: (0, 0)),
            pl.BlockSpec((1, d_edge), lambda i: (0, 0)),
            pl.BlockSpec((d_edge, w_numel), lambda i: (0, 0)),
            pl.BlockSpec((1, w_numel), lambda i: (0, 0)),
        ],
        out_specs=pl.BlockSpec((_E_TILE, _D_OUT), lambda i: (i, 0)),
        out_shape=jax.ShapeDtypeStruct((E, _D_OUT), jnp.float32),
        interpret=interpret,
    )(edge_attr, x, edge_sh, W1, b1.reshape(1, -1), W2, b2.reshape(1, -1))


def kernel(node_attr, edge_index, edge_attr, edge_sh, W1, b1, W2, b2):
    edge_src = edge_index[0]
    edge_dst = edge_index[1]
    x = node_attr[edge_dst]
    tp = _edge_compute(edge_attr, x, edge_sh, W1, b1, W2, b2)
    n = node_attr.shape[0]
    sums = jax.ops.segment_sum(tp, edge_src, num_segments=n)
    cnt = jax.ops.segment_sum(jnp.ones((tp.shape[0],), jnp.float32), edge_src,
                              num_segments=n)
    return sums / jnp.maximum(cnt, 1.0)[:, None] + node_attr


# speedup vs baseline: 2.0124x; 2.0124x over previous
import functools

import jax
import jax.numpy as jnp
from jax.experimental import pallas as pl
from jax.experimental.pallas import tpu as pltpu

_D_NODE = 48
_D_OUT = 48
_D_PAD = 64
_INV_SQRT_FAN = 1.0 / (48.0 ** 0.5)

_E_TILE = 256


def _edge_body(a_ref, x_ref, sh_ref, w1_ref, b1_ref, w2p_ref, s_ref, tp_ref):
    h = jnp.dot(a_ref[...], w1_ref[...], preferred_element_type=jnp.float32)
    h = jnp.maximum(h + b1_ref[...], 0.0)
    sc = sh_ref[...] * _INV_SQRT_FAN
    hp = jnp.concatenate([h * sc, sc], axis=1)
    wr = jnp.dot(hp.astype(jnp.bfloat16), w2p_ref[...],
                 preferred_element_type=jnp.float32
                 ).astype(jnp.bfloat16)
    et = a_ref.shape[0]
    xp = jnp.concatenate(
        [x_ref[...], jnp.zeros((et, _D_PAD - _D_NODE), jnp.float32)], axis=1)
    xt = jnp.tile(xp.astype(jnp.bfloat16), (1, _D_OUT))
    tp = jnp.dot(wr * xt, s_ref[...], preferred_element_type=jnp.float32)
    lane = jax.lax.broadcasted_iota(jnp.int32, tp.shape, 1)
    tp_ref[...] = jnp.where(lane == _D_NODE, 1.0, tp)


def _edge_compute(edge_attr, x, edge_sh, W1, b1, W2P, S64, interpret=False):
    E, d_edge = edge_attr.shape
    kw = _D_OUT * _D_PAD
    grid = (E // _E_TILE,)
    return pl.pallas_call(
        _edge_body,
        grid=grid,
        in_specs=[
            pl.BlockSpec((_E_TILE, d_edge), lambda i: (i, 0)),
            pl.BlockSpec((_E_TILE, _D_NODE), lambda i: (i, 0)),
            pl.BlockSpec((_E_TILE, 1), lambda i: (i, 0)),
            pl.BlockSpec((d_edge, d_edge), lambda i: (0, 0)),
            pl.BlockSpec((1, d_edge), lambda i: (0, 0)),
            pl.BlockSpec((d_edge + 1, kw), lambda i: (0, 0)),
            pl.BlockSpec((kw, _D_PAD), lambda i: (0, 0)),
        ],
        out_specs=pl.BlockSpec((_E_TILE, _D_PAD), lambda i: (i, 0)),
        out_shape=jax.ShapeDtypeStruct((E, _D_PAD), jnp.float32),
        interpret=interpret,
    )(edge_attr, x, edge_sh, W1, b1.reshape(1, -1), W2P, S64)


def _prep_weights(W2, b2):
    d_hid = W2.shape[0]
    w2t = W2.reshape(d_hid, _D_NODE, _D_OUT).transpose(0, 2, 1)
    b2t = b2.reshape(_D_NODE, _D_OUT).T[None]
    w2p = jnp.concatenate([w2t, b2t], axis=0)
    w2p = jnp.pad(w2p, ((0, 0), (0, 0), (0, _D_PAD - _D_NODE)))
    W2P = w2p.reshape(d_hid + 1, _D_OUT * _D_PAD).astype(jnp.bfloat16)
    j = jnp.arange(_D_OUT * _D_PAD)
    S64 = ((j[:, None] // _D_PAD == jnp.arange(_D_PAD)[None, :])
           & (j[:, None] % _D_PAD < _D_NODE)).astype(jnp.bfloat16)
    return W2P, S64


def kernel(node_attr, edge_index, edge_attr, edge_sh, W1, b1, W2, b2):
    edge_src = edge_index[0]
    edge_dst = edge_index[1]
    W2P, S64 = _prep_weights(W2, b2)
    x = node_attr[edge_dst]
    tp64 = _edge_compute(edge_attr, x, edge_sh, W1, b1, W2P, S64)
    n = node_attr.shape[0]
    sums = jax.ops.segment_sum(tp64, edge_src, num_segments=n)
    return sums[:, :_D_OUT] / jnp.maximum(sums[:, _D_NODE:_D_NODE + 1], 1.0) + node_attr


# speedup vs baseline: 3.6870x; 1.8322x over previous
import functools

import jax
import jax.numpy as jnp
from jax import lax
from jax.experimental import pallas as pl
from jax.experimental.pallas import tpu as pltpu
from jax.experimental.pallas import tpu_sc as plsc

_NC = 2
_NS = 16
_NW = _NC * _NS

_D_NODE = 48
_D_OUT = 48
_D_PAD = 64
_D_ROW = 128
_INV_SQRT_FAN = 1.0 / (48.0 ** 0.5)

_E_TILE = 256


def _edge_body(a_ref, x_ref, sh_ref, w1_ref, b1_ref, w2p_ref, s_ref, tp_ref):
    h = jnp.dot(a_ref[...], w1_ref[...], preferred_element_type=jnp.float32)
    h = jnp.maximum(h + b1_ref[...], 0.0)
    sc = sh_ref[...] * _INV_SQRT_FAN
    hp = jnp.concatenate([h * sc, sc], axis=1)
    wr = jnp.dot(hp.astype(jnp.bfloat16), w2p_ref[...],
                 preferred_element_type=jnp.float32
                 ).astype(jnp.bfloat16)
    xr = x_ref[...][:, :_D_PAD]
    lane64 = jax.lax.broadcasted_iota(jnp.int32, xr.shape, 1)
    xp = jnp.where(lane64 < _D_NODE, xr, 0.0)
    xt = jnp.tile(xp.astype(jnp.bfloat16), (1, _D_OUT))
    tp = jnp.dot(wr * xt, s_ref[...], preferred_element_type=jnp.float32)
    lane = jax.lax.broadcasted_iota(jnp.int32, tp.shape, 1)
    tp_ref[...] = jnp.where(lane == _D_NODE, 1.0, tp)


def _edge_compute(edge_attr, x, edge_sh, W1, b1, W2P, S64, interpret=False):
    E, d_edge = edge_attr.shape
    kw = _D_OUT * _D_PAD
    grid = (E // _E_TILE,)
    return pl.pallas_call(
        _edge_body,
        grid=grid,
        in_specs=[
            pl.BlockSpec((_E_TILE, d_edge), lambda i: (i, 0)),
            pl.BlockSpec((_E_TILE, 128), lambda i: (i, 0)),
            pl.BlockSpec((_E_TILE, 1), lambda i: (i, 0)),
            pl.BlockSpec((d_edge, d_edge), lambda i: (0, 0)),
            pl.BlockSpec((1, d_edge), lambda i: (0, 0)),
            pl.BlockSpec((d_edge + 1, kw), lambda i: (0, 0)),
            pl.BlockSpec((kw, _D_ROW), lambda i: (0, 0)),
        ],
        out_specs=pl.BlockSpec((_E_TILE, _D_ROW), lambda i: (i, 0)),
        out_shape=jax.ShapeDtypeStruct((E, _D_ROW), jnp.float32),
        interpret=interpret,
    )(edge_attr, x, edge_sh, W1, b1.reshape(1, -1), W2P, S64)


def _prep_weights(W2, b2):
    d_hid = W2.shape[0]
    w2t = W2.reshape(d_hid, _D_NODE, _D_OUT).transpose(0, 2, 1)
    b2t = b2.reshape(_D_NODE, _D_OUT).T[None]
    w2p = jnp.concatenate([w2t, b2t], axis=0)
    w2p = jnp.pad(w2p, ((0, 0), (0, 0), (0, _D_PAD - _D_NODE)))
    W2P = w2p.reshape(d_hid + 1, _D_OUT * _D_PAD).astype(jnp.bfloat16)
    j = jnp.arange(_D_OUT * _D_PAD)
    S64 = ((j[:, None] // _D_PAD == jnp.arange(_D_ROW)[None, :])
           & (j[:, None] % _D_PAD < _D_NODE)).astype(jnp.bfloat16)
    return W2P, S64


def _sc_gather(node128, dst3):
    n, d = node128.shape
    nw, nchunk, cw = dst3.shape
    epw = nchunk * cw
    half = nchunk // 2
    mesh = plsc.VectorSubcoreMesh(core_axis_name="c", subcore_axis_name="s")

    @functools.partial(
        pl.kernel, mesh=mesh,
        out_type=jax.ShapeDtypeStruct((nw * epw, d), jnp.float32),
        scratch_types=[
            pltpu.VMEM((nchunk, cw), jnp.int32),
            pltpu.VMEM((half * cw, d), jnp.float32),
            pltpu.SemaphoreType.DMA,
        ],
    )
    def g(node_hbm, idx_hbm, x_hbm, idx_v, rows_v, sem):
        wid = lax.axis_index("c") * _NS + lax.axis_index("s")
        base = wid * epw
        pltpu.sync_copy(idx_hbm.at[wid], idx_v)
        for hh in range(2):
            cps = [pltpu.async_copy(node_hbm.at[idx_v.at[hh * half + j]],
                                    rows_v.at[pl.ds(j * cw, cw)], sem)
                   for j in range(half)]
            for cp in cps:
                cp.wait()
            pltpu.sync_copy(rows_v,
                            x_hbm.at[pl.ds(base + hh * half * cw, half * cw)])

    return g(node128, dst3)


def _sc_scatter(tp64, src3, zinit):
    e, d = tp64.shape
    nw, nchunk, cw = src3.shape
    epw = nchunk * cw
    n = zinit.shape[0]
    rps = n // _NS
    mesh = plsc.VectorSubcoreMesh(core_axis_name="c", subcore_axis_name="s")

    @functools.partial(
        pl.kernel, mesh=mesh,
        out_type=jax.ShapeDtypeStruct((_NC, n, d), jnp.float32),
        scratch_types=[
            pltpu.VMEM((nchunk, cw), jnp.int32),
            pltpu.VMEM((epw // 2, d), jnp.float32),
            pltpu.VMEM_SHARED((n, d), jnp.float32),
        ],
    )
    def k(tp_hbm, idx_hbm, z_hbm, out_hbm, idx_v, rows_v, acc_sh):
        c = lax.axis_index("c")
        s = lax.axis_index("s")
        wid = c * _NS + s
        base = wid * epw
        pltpu.sync_copy(z_hbm.at[pl.ds(s * rps, rps)],
                        acc_sh.at[pl.ds(s * rps, rps)])
        plsc.subcore_barrier()
        pltpu.sync_copy(idx_hbm.at[wid], idx_v)
        half = nchunk // 2
        for hh in range(2):
            pltpu.sync_copy(
                tp_hbm.at[pl.ds(base + hh * half * cw, half * cw)], rows_v)
            for j in range(half):
                pltpu.sync_copy(rows_v.at[pl.ds(j * cw, cw)],
                                acc_sh.at[idx_v.at[hh * half + j]], add=True)
        plsc.subcore_barrier()
        pltpu.sync_copy(acc_sh.at[pl.ds(s * rps, rps)],
                        out_hbm.at[c].at[pl.ds(s * rps, rps)])

    return k(tp64, src3, zinit)


def _combine_body(p_ref, na_ref, o_ref):
    ps = p_ref[0] + p_ref[1]
    cnt = jnp.maximum(ps[:, _D_NODE:_D_NODE + 1], 1.0)
    o_ref[...] = ps[:, :_D_OUT] / cnt + na_ref[...]


def _combine(partials, node_attr, interpret=False):
    n, d = node_attr.shape
    nt = 1024
    return pl.pallas_call(
        _combine_body,
        grid=(n // nt,),
        in_specs=[
            pl.BlockSpec((_NC, nt, _D_ROW), lambda i: (0, i, 0)),
            pl.BlockSpec((nt, d), lambda i: (i, 0)),
        ],
        out_specs=pl.BlockSpec((nt, d), lambda i: (i, 0)),
        out_shape=jax.ShapeDtypeStruct((n, d), jnp.float32),
        interpret=interpret,
    )(partials, node_attr)


def kernel(node_attr, edge_index, edge_attr, edge_sh, W1, b1, W2, b2):
    e = edge_index.shape[1]
    nchunk = e // (_NW * 128)
    src3 = edge_index[0].reshape(_NW, nchunk, 128)
    dst3 = edge_index[1].reshape(_NW, nchunk, 128)
    W2P, S64 = _prep_weights(W2, b2)
    node128 = jnp.pad(node_attr, ((0, 0), (0, 128 - node_attr.shape[1])))
    x = _sc_gather(node128, dst3)
    tp64 = _edge_compute(edge_attr, x, edge_sh, W1, b1, W2P, S64)
    n = node_attr.shape[0]
    zinit = jnp.zeros((n, _D_ROW), jnp.float32)
    partials = _sc_scatter(tp64, src3, zinit)
    return _combine(partials, node_attr)


# speedup vs baseline: 4.1050x; 1.1134x over previous
import functools

import jax
import jax.numpy as jnp
from jax import lax
from jax.experimental import pallas as pl
from jax.experimental.pallas import tpu as pltpu
from jax.experimental.pallas import tpu_sc as plsc

_NC = 2
_NS = 16
_NW = _NC * _NS

_D_NODE = 48
_D_OUT = 48
_D_PAD = 64
_D_ROW = 128
_INV_SQRT_FAN = 1.0 / (48.0 ** 0.5)

_E_TILE = 2048


def _edge_body(a_ref, x_ref, sh_ref, w1_ref, b1_ref, w2p_ref, s_ref, tp_ref):
    h = jnp.dot(a_ref[...], w1_ref[...], preferred_element_type=jnp.float32)
    h = jnp.maximum(h + b1_ref[...], 0.0)
    sc = sh_ref[...] * _INV_SQRT_FAN
    hp = jnp.concatenate([h * sc, sc], axis=1).astype(jnp.bfloat16)
    xr = x_ref[...][:, :_D_PAD]
    lane64 = jax.lax.broadcasted_iota(jnp.int32, xr.shape, 1)
    xp = jnp.where(lane64 < _D_NODE, xr, 0.0).astype(jnp.bfloat16)
    xt4 = jnp.tile(xp, (1, 4))
    et = a_ref.shape[0]
    acc = jnp.zeros((et, _D_ROW), jnp.float32)
    nchunks = w2p_ref.shape[1] // 256
    for j in range(nchunks):
        wr_j = jnp.dot(hp, w2p_ref[:, j * 256:(j + 1) * 256],
                       preferred_element_type=jnp.float32).astype(jnp.bfloat16)
        acc = acc + jnp.dot(wr_j * xt4, s_ref[j * 256:(j + 1) * 256, :],
                            preferred_element_type=jnp.float32)
    lane = jax.lax.broadcasted_iota(jnp.int32, acc.shape, 1)
    tp_ref[...] = jnp.where(lane == _D_NODE, 1.0, acc)


def _edge_compute(edge_attr, x, edge_sh, W1, b1, W2P, S64, interpret=False):
    E, d_edge = edge_attr.shape
    kw = _D_OUT * _D_PAD
    grid = (E // _E_TILE,)
    return pl.pallas_call(
        _edge_body,
        grid=grid,
        in_specs=[
            pl.BlockSpec((_E_TILE, d_edge), lambda i: (i, 0)),
            pl.BlockSpec((_E_TILE, 128), lambda i: (i, 0)),
            pl.BlockSpec((_E_TILE, 1), lambda i: (i, 0)),
            pl.BlockSpec((d_edge, d_edge), lambda i: (0, 0)),
            pl.BlockSpec((1, d_edge), lambda i: (0, 0)),
            pl.BlockSpec((d_edge + 1, kw), lambda i: (0, 0)),
            pl.BlockSpec((kw, _D_ROW), lambda i: (0, 0)),
        ],
        out_specs=pl.BlockSpec((_E_TILE, _D_ROW), lambda i: (i, 0)),
        out_shape=jax.ShapeDtypeStruct((E, _D_ROW), jnp.float32),
        interpret=interpret,
    )(edge_attr.astype(jnp.bfloat16), x, edge_sh,
      W1.astype(jnp.bfloat16), b1.reshape(1, -1), W2P, S64)


def _prep_weights(W2, b2):
    d_hid = W2.shape[0]
    w2t = W2.reshape(d_hid, _D_NODE, _D_OUT).transpose(0, 2, 1)
    b2t = b2.reshape(_D_NODE, _D_OUT).T[None]
    w2p = jnp.concatenate([w2t, b2t], axis=0)
    w2p = jnp.pad(w2p, ((0, 0), (0, 0), (0, _D_PAD - _D_NODE)))
    W2P = w2p.reshape(d_hid + 1, _D_OUT * _D_PAD).astype(jnp.bfloat16)
    j = jnp.arange(_D_OUT * _D_PAD)
    S64 = ((j[:, None] // _D_PAD == jnp.arange(_D_ROW)[None, :])
           & (j[:, None] % _D_PAD < _D_NODE)).astype(jnp.bfloat16)
    return W2P, S64


def _sc_gather(node128, dst3):
    n, d = node128.shape
    nw, nchunk, cw = dst3.shape
    epw = nchunk * cw
    mesh = plsc.VectorSubcoreMesh(core_axis_name="c", subcore_axis_name="s")

    @functools.partial(
        pl.kernel, mesh=mesh,
        out_type=jax.ShapeDtypeStruct((nw * epw, d), jnp.float32),
        scratch_types=[
            pltpu.VMEM((nchunk, cw), jnp.int32),
            pltpu.VMEM((epw // 2, d), jnp.float32),
            pltpu.SemaphoreType.DMA,
        ],
    )
    def g(node_hbm, idx_hbm, x_hbm, idx_v, rows_v, sem):
        wid = lax.axis_index("c") * _NS + lax.axis_index("s")
        base = wid * epw
        pltpu.sync_copy(idx_hbm.at[wid], idx_v)
        half = nchunk // 2
        for hh in range(2):
            cps = [pltpu.async_copy(node_hbm.at[idx_v.at[hh * half + j]],
                                    rows_v.at[pl.ds(j * cw, cw)], sem)
                   for j in range(half)]
            for cp in cps:
                cp.wait()
            pltpu.sync_copy(rows_v,
                            x_hbm.at[pl.ds(base + hh * half * cw, half * cw)])

    return g(node128, dst3)


def _sc_scatter(tp64, src3, zinit):
    e, d = tp64.shape
    nw, nchunk, cw = src3.shape
    epw = nchunk * cw
    n = zinit.shape[0]
    rps = n // _NS
    mesh = plsc.VectorSubcoreMesh(core_axis_name="c", subcore_axis_name="s")

    @functools.partial(
        pl.kernel, mesh=mesh,
        out_type=jax.ShapeDtypeStruct((_NC, n, d), jnp.float32),
        scratch_types=[
            pltpu.VMEM((nchunk, cw), jnp.int32),
            pltpu.VMEM((epw // 2, d), jnp.float32),
            pltpu.VMEM_SHARED((n, d), jnp.float32),
        ],
    )
    def k(tp_hbm, idx_hbm, z_hbm, out_hbm, idx_v, rows_v, acc_sh):
        c = lax.axis_index("c")
        s = lax.axis_index("s")
        wid = c * _NS + s
        base = wid * epw
        pltpu.sync_copy(z_hbm.at[pl.ds(s * rps, rps)],
                        acc_sh.at[pl.ds(s * rps, rps)])
        plsc.subcore_barrier()
        pltpu.sync_copy(idx_hbm.at[wid], idx_v)
        half = nchunk // 2
        for hh in range(2):
            pltpu.sync_copy(
                tp_hbm.at[pl.ds(base + hh * half * cw, half * cw)], rows_v)
            for j in range(half):
                pltpu.sync_copy(rows_v.at[pl.ds(j * cw, cw)],
                                acc_sh.at[idx_v.at[hh * half + j]], add=True)
        plsc.subcore_barrier()
        pltpu.sync_copy(acc_sh.at[pl.ds(s * rps, rps)],
                        out_hbm.at[c].at[pl.ds(s * rps, rps)])

    return k(tp64, src3, zinit)


def _combine_body(p_ref, na_ref, o_ref):
    ps = p_ref[0] + p_ref[1]
    cnt = jnp.maximum(ps[:, _D_NODE:_D_NODE + 1], 1.0)
    o_ref[...] = ps[:, :_D_OUT] / cnt + na_ref[...]


def _combine(partials, node_attr, interpret=False):
    n, d = node_attr.shape
    nt = 1024
    return pl.pallas_call(
        _combine_body,
        grid=(n // nt,),
        in_specs=[
            pl.BlockSpec((_NC, nt, _D_ROW), lambda i: (0, i, 0)),
            pl.BlockSpec((nt, d), lambda i: (i, 0)),
        ],
        out_specs=pl.BlockSpec((nt, d), lambda i: (i, 0)),
        out_shape=jax.ShapeDtypeStruct((n, d), jnp.float32),
        interpret=interpret,
    )(partials, node_attr)


def kernel(node_attr, edge_index, edge_attr, edge_sh, W1, b1, W2, b2):
    e = edge_index.shape[1]
    nchunk = e // (_NW * 128)
    src3 = edge_index[0].reshape(_NW, nchunk, 128)
    dst3 = edge_index[1].reshape(_NW, nchunk, 128)
    W2P, S64 = _prep_weights(W2, b2)
    node128 = jnp.pad(node_attr, ((0, 0), (0, 128 - node_attr.shape[1])))
    x = _sc_gather(node128, dst3)
    tp64 = _edge_compute(edge_attr, x, edge_sh, W1, b1, W2P, S64)
    n = node_attr.shape[0]
    zinit = jnp.zeros((n, _D_ROW), jnp.float32)
    partials = _sc_scatter(tp64, src3, zinit)
    return _combine(partials, node_attr)


# speedup vs baseline: 4.1559x; 1.0124x over previous
import functools

import jax
import jax.numpy as jnp
from jax import lax
from jax.experimental import pallas as pl
from jax.experimental.pallas import tpu as pltpu
from jax.experimental.pallas import tpu_sc as plsc

_NC = 2
_NS = 16
_NW = _NC * _NS

_D_NODE = 48
_D_OUT = 48
_D_PAD = 64
_D_ROW = 128
_INV_SQRT_FAN = 1.0 / (48.0 ** 0.5)

_E_TILE = 2048


def _edge_body(a_ref, x_ref, sh_ref, w1_ref, b1_ref, w2p_ref, s_ref, tp_ref):
    h = jnp.dot(a_ref[...], w1_ref[...], preferred_element_type=jnp.float32)
    h = jnp.maximum(h + b1_ref[...], 0.0)
    sc = sh_ref[...] * _INV_SQRT_FAN
    hp = jnp.concatenate([h * sc, sc], axis=1).astype(jnp.bfloat16)
    xr = x_ref[...][:, :_D_PAD]
    lane64 = jax.lax.broadcasted_iota(jnp.int32, xr.shape, 1)
    xp = jnp.where(lane64 < _D_NODE, xr, 0.0).astype(jnp.bfloat16)
    xt4 = jnp.tile(xp, (1, 4))
    et = a_ref.shape[0]
    acc = jnp.zeros((et, _D_ROW), jnp.float32)
    nchunks = w2p_ref.shape[1] // 256
    for j in range(nchunks):
        wr_j = jnp.dot(hp, w2p_ref[:, j * 256:(j + 1) * 256],
                       preferred_element_type=jnp.float32).astype(jnp.bfloat16)
        acc = acc + jnp.dot(wr_j * xt4, s_ref[j * 256:(j + 1) * 256, :],
                            preferred_element_type=jnp.float32)
    lane = jax.lax.broadcasted_iota(jnp.int32, acc.shape, 1)
    tp_ref[...] = jnp.where(lane == _D_NODE, 1.0, acc)


def _edge_compute(edge_attr, x, edge_sh, W1, b1, W2P, S64, interpret=False):
    E, d_edge = edge_attr.shape
    kw = _D_OUT * _D_PAD
    grid = (E // _E_TILE,)
    return pl.pallas_call(
        _edge_body,
        grid=grid,
        in_specs=[
            pl.BlockSpec((_E_TILE, d_edge), lambda i: (i, 0)),
            pl.BlockSpec((_E_TILE, 128), lambda i: (i, 0)),
            pl.BlockSpec((_E_TILE, 1), lambda i: (i, 0)),
            pl.BlockSpec((d_edge, d_edge), lambda i: (0, 0)),
            pl.BlockSpec((1, d_edge), lambda i: (0, 0)),
            pl.BlockSpec((d_edge + 1, kw), lambda i: (0, 0)),
            pl.BlockSpec((kw, _D_ROW), lambda i: (0, 0)),
        ],
        out_specs=pl.BlockSpec((_E_TILE, _D_ROW), lambda i: (i, 0)),
        out_shape=jax.ShapeDtypeStruct((E, _D_ROW), jnp.float32),
        interpret=interpret,
    )(edge_attr.astype(jnp.bfloat16), x, edge_sh,
      W1.astype(jnp.bfloat16), b1.reshape(1, -1), W2P, S64)


def _prep_weights(W2, b2):
    d_hid = W2.shape[0]
    w2t = W2.reshape(d_hid, _D_NODE, _D_OUT).transpose(0, 2, 1)
    b2t = b2.reshape(_D_NODE, _D_OUT).T[None]
    w2p = jnp.concatenate([w2t, b2t], axis=0)
    w2p = jnp.pad(w2p, ((0, 0), (0, 0), (0, _D_PAD - _D_NODE)))
    W2P = w2p.reshape(d_hid + 1, _D_OUT * _D_PAD).astype(jnp.bfloat16)
    j = jnp.arange(_D_OUT * _D_PAD)
    S64 = ((j[:, None] // _D_PAD == jnp.arange(_D_ROW)[None, :])
           & (j[:, None] % _D_PAD < _D_NODE)).astype(jnp.bfloat16)
    return W2P, S64


def _sc_gather(node128, dst3):
    n, d = node128.shape
    nw, nchunk, cw = dst3.shape
    epw = nchunk * cw
    nbuf = 6
    mesh = plsc.VectorSubcoreMesh(core_axis_name="c", subcore_axis_name="s")

    @functools.partial(
        pl.kernel, mesh=mesh,
        out_type=jax.ShapeDtypeStruct((nw * epw, d), jnp.float32),
        scratch_types=[
            pltpu.VMEM((nchunk, cw), jnp.int32),
            pltpu.VMEM((nbuf * cw, d), jnp.float32),
            pltpu.SemaphoreType.DMA,
            pltpu.SemaphoreType.DMA,
        ],
    )
    def g(node_hbm, idx_hbm, x_hbm, idx_v, rows_v, gsem, wsem):
        wid = lax.axis_index("c") * _NS + lax.axis_index("s")
        base = wid * epw
        pltpu.sync_copy(idx_hbm.at[wid], idx_v)
        gs = {}
        for j in range(nbuf):
            gs[j] = pltpu.async_copy(node_hbm.at[idx_v.at[j]],
                                     rows_v.at[pl.ds(j * cw, cw)], gsem)
        ws = {}
        for j in range(nchunk):
            b = j % nbuf
            gs[j].wait()
            ws[j] = pltpu.async_copy(rows_v.at[pl.ds(b * cw, cw)],
                                     x_hbm.at[pl.ds(base + j * cw, cw)], wsem)
            nj = j + nbuf
            if nj < nchunk:
                ws[j].wait()
                gs[nj] = pltpu.async_copy(node_hbm.at[idx_v.at[nj]],
                                          rows_v.at[pl.ds(b * cw, cw)], gsem)
        for j in range(nchunk - nbuf, nchunk):
            ws[j].wait()

    return g(node128, dst3)


def _sc_scatter(tp64, src3, zinit):
    e, d = tp64.shape
    nw, nchunk, cw = src3.shape
    epw = nchunk * cw
    n = zinit.shape[0]
    rps = n // _NS
    qrows = 2 * cw
    nq = epw // qrows
    mesh = plsc.VectorSubcoreMesh(core_axis_name="c", subcore_axis_name="s")

    @functools.partial(
        pl.kernel, mesh=mesh,
        out_type=jax.ShapeDtypeStruct((_NC, n, d), jnp.float32),
        scratch_types=[
            pltpu.VMEM((nchunk, cw), jnp.int32),
            pltpu.VMEM((2, qrows, d), jnp.float32),
            pltpu.VMEM_SHARED((n, d), jnp.float32),
            pltpu.SemaphoreType.DMA,
        ],
    )
    def k(tp_hbm, idx_hbm, z_hbm, out_hbm, idx_v, rows_v, acc_sh, lsem):
        c = lax.axis_index("c")
        s = lax.axis_index("s")
        wid = c * _NS + s
        base = wid * epw
        lds = {0: pltpu.async_copy(tp_hbm.at[pl.ds(base, qrows)],
                                   rows_v.at[0], lsem)}
        pltpu.sync_copy(idx_hbm.at[wid], idx_v)
        pltpu.sync_copy(z_hbm.at[pl.ds(s * rps, rps)],
                        acc_sh.at[pl.ds(s * rps, rps)])
        plsc.subcore_barrier()
        for q in range(nq):
            lds[q].wait()
            if q + 1 < nq:
                lds[q + 1] = pltpu.async_copy(
                    tp_hbm.at[pl.ds(base + (q + 1) * qrows, qrows)],
                    rows_v.at[(q + 1) % 2], lsem)
            for jj in range(2):
                pltpu.sync_copy(rows_v.at[q % 2].at[pl.ds(jj * cw, cw)],
                                acc_sh.at[idx_v.at[2 * q + jj]], add=True)
        plsc.subcore_barrier()
        pltpu.sync_copy(acc_sh.at[pl.ds(s * rps, rps)],
                        out_hbm.at[c].at[pl.ds(s * rps, rps)])

    return k(tp64, src3, zinit)


def _combine_body(p_ref, na_ref, o_ref):
    ps = p_ref[0] + p_ref[1]
    cnt = jnp.maximum(ps[:, _D_NODE:_D_NODE + 1], 1.0)
    o_ref[...] = ps[:, :_D_OUT] / cnt + na_ref[...]


def _combine(partials, node_attr, interpret=False):
    n, d = node_attr.shape
    nt = 1024
    return pl.pallas_call(
        _combine_body,
        grid=(n // nt,),
        in_specs=[
            pl.BlockSpec((_NC, nt, _D_ROW), lambda i: (0, i, 0)),
            pl.BlockSpec((nt, d), lambda i: (i, 0)),
        ],
        out_specs=pl.BlockSpec((nt, d), lambda i: (i, 0)),
        out_shape=jax.ShapeDtypeStruct((n, d), jnp.float32),
        interpret=interpret,
    )(partials, node_attr)


def kernel(node_attr, edge_index, edge_attr, edge_sh, W1, b1, W2, b2):
    e = edge_index.shape[1]
    nchunk = e // (_NW * 128)
    src3 = edge_index[0].reshape(_NW, nchunk, 128)
    dst3 = edge_index[1].reshape(_NW, nchunk, 128)
    W2P, S64 = _prep_weights(W2, b2)
    node128 = jnp.pad(node_attr, ((0, 0), (0, 128 - node_attr.shape[1])))
    x = _sc_gather(node128, dst3)
    tp64 = _edge_compute(edge_attr, x, edge_sh, W1, b1, W2P, S64)
    n = node_attr.shape[0]
    zinit = jnp.zeros((n, _D_ROW), jnp.float32)
    partials = _sc_scatter(tp64, src3, zinit)
    return _combine(partials, node_attr)


# speedup vs baseline: 4.2485x; 1.0223x over previous
import functools

import jax
import jax.numpy as jnp
from jax import lax
from jax.experimental import pallas as pl
from jax.experimental.pallas import tpu as pltpu
from jax.experimental.pallas import tpu_sc as plsc

_NC = 2
_NS = 16
_NW = _NC * _NS

_D_NODE = 48
_D_OUT = 48
_D_PAD = 64
_D_ROW = 128
_INV_SQRT_FAN = 1.0 / (48.0 ** 0.5)

_E_TILE = 2048


def _edge_body(a_ref, x_ref, sh_ref, w1_ref, b1_ref, w2p_ref, s_ref, tp_ref):
    et = a_ref.shape[0]
    h = jnp.dot(a_ref[...].astype(jnp.bfloat16), w1_ref[...],
                preferred_element_type=jnp.float32)
    h = jnp.maximum(h + b1_ref[...], 0.0)
    sc = jnp.transpose(sh_ref[...]) * _INV_SQRT_FAN
    hp = jnp.concatenate([h * sc, sc], axis=1).astype(jnp.bfloat16)
    xr = x_ref[...][:, :_D_PAD]
    lane64 = jax.lax.broadcasted_iota(jnp.int32, xr.shape, 1)
    xp = jnp.where(lane64 < _D_NODE, xr, 0.0).astype(jnp.bfloat16)
    xt4 = jnp.tile(xp, (1, 4))
    acc = jnp.zeros((et, _D_ROW), jnp.float32)
    nchunks = w2p_ref.shape[1] // 256
    for j in range(nchunks):
        wr_j = jnp.dot(hp, w2p_ref[:, j * 256:(j + 1) * 256],
                       preferred_element_type=jnp.float32).astype(jnp.bfloat16)
        acc = acc + jnp.dot(wr_j * xt4, s_ref[j * 256:(j + 1) * 256, :],
                            preferred_element_type=jnp.float32)
    lane = jax.lax.broadcasted_iota(jnp.int32, acc.shape, 1)
    tp_ref[...] = jnp.where(lane == _D_NODE, 1.0, acc)


def _edge_compute(edge_attr, x, edge_sh, W1, b1, W2P, S64, interpret=False):
    E, d_edge = edge_attr.shape
    kw = _D_OUT * _D_PAD
    grid = (E // _E_TILE,)
    return pl.pallas_call(
        _edge_body,
        grid=grid,
        in_specs=[
            pl.BlockSpec((_E_TILE, d_edge), lambda i: (i, 0)),
            pl.BlockSpec((_E_TILE, 128), lambda i: (i, 0)),
            pl.BlockSpec((1, _E_TILE), lambda i: (0, i)),
            pl.BlockSpec((d_edge, d_edge), lambda i: (0, 0)),
            pl.BlockSpec((1, d_edge), lambda i: (0, 0)),
            pl.BlockSpec((d_edge + 1, kw), lambda i: (0, 0)),
            pl.BlockSpec((kw, _D_ROW), lambda i: (0, 0)),
        ],
        out_specs=pl.BlockSpec((_E_TILE, _D_ROW), lambda i: (i, 0)),
        out_shape=jax.ShapeDtypeStruct((E, _D_ROW), jnp.float32),
        interpret=interpret,
    )(edge_attr, x, edge_sh.reshape(1, E),
      W1.astype(jnp.bfloat16), b1.reshape(1, -1), W2P, S64)


def _prep_weights(W2, b2):
    d_hid = W2.shape[0]
    w2t = W2.reshape(d_hid, _D_NODE, _D_OUT).transpose(0, 2, 1)
    b2t = b2.reshape(_D_NODE, _D_OUT).T[None]
    w2p = jnp.concatenate([w2t, b2t], axis=0)
    w2p = jnp.pad(w2p, ((0, 0), (0, 0), (0, _D_PAD - _D_NODE)))
    W2P = w2p.reshape(d_hid + 1, _D_OUT * _D_PAD).astype(jnp.bfloat16)
    j = jnp.arange(_D_OUT * _D_PAD)
    S64 = ((j[:, None] // _D_PAD == jnp.arange(_D_ROW)[None, :])
           & (j[:, None] % _D_PAD < _D_NODE)).astype(jnp.bfloat16)
    return W2P, S64


def _sc_gather(node128, dst3):
    n, d = node128.shape
    nw, nchunk, cw = dst3.shape
    epw = nchunk * cw
    nbuf = 6
    mesh = plsc.VectorSubcoreMesh(core_axis_name="c", subcore_axis_name="s")

    @functools.partial(
        pl.kernel, mesh=mesh,
        out_type=jax.ShapeDtypeStruct((nw * epw, d), jnp.float32),
        scratch_types=[
            pltpu.VMEM((nchunk, cw), jnp.int32),
            pltpu.VMEM((nbuf * cw, d), jnp.float32),
            pltpu.SemaphoreType.DMA,
            pltpu.SemaphoreType.DMA,
        ],
    )
    def g(node_hbm, idx_hbm, x_hbm, idx_v, rows_v, gsem, wsem):
        wid = lax.axis_index("c") * _NS + lax.axis_index("s")
        base = wid * epw
        pltpu.sync_copy(idx_hbm.at[wid], idx_v)
        gs = {}
        for j in range(nbuf):
            gs[j] = pltpu.async_copy(node_hbm.at[idx_v.at[j]],
                                     rows_v.at[pl.ds(j * cw, cw)], gsem)
        ws = {}
        for j in range(nchunk):
            b = j % nbuf
            gs[j].wait()
            ws[j] = pltpu.async_copy(rows_v.at[pl.ds(b * cw, cw)],
                                     x_hbm.at[pl.ds(base + j * cw, cw)], wsem)
            nj = j + nbuf
            if nj < nchunk:
                ws[j].wait()
                gs[nj] = pltpu.async_copy(node_hbm.at[idx_v.at[nj]],
                                          rows_v.at[pl.ds(b * cw, cw)], gsem)
        for j in range(nchunk - nbuf, nchunk):
            ws[j].wait()

    return g(node128, dst3)


def _sc_scatter(tp64, src3, zinit):
    e, d = tp64.shape
    nw, nchunk, cw = src3.shape
    epw = nchunk * cw
    n = zinit.shape[0]
    rps = n // _NS
    qrows = 2 * cw
    nq = epw // qrows
    mesh = plsc.VectorSubcoreMesh(core_axis_name="c", subcore_axis_name="s")

    @functools.partial(
        pl.kernel, mesh=mesh,
        out_type=jax.ShapeDtypeStruct((_NC, n, d), jnp.float32),
        scratch_types=[
            pltpu.VMEM((nchunk, cw), jnp.int32),
            pltpu.VMEM((2, qrows, d), jnp.float32),
            pltpu.VMEM_SHARED((n, d), jnp.float32),
            pltpu.SemaphoreType.DMA,
        ],
    )
    def k(tp_hbm, idx_hbm, z_hbm, out_hbm, idx_v, rows_v, acc_sh, lsem):
        c = lax.axis_index("c")
        s = lax.axis_index("s")
        wid = c * _NS + s
        base = wid * epw
        lds = {0: pltpu.async_copy(tp_hbm.at[pl.ds(base, qrows)],
                                   rows_v.at[0], lsem)}
        pltpu.sync_copy(idx_hbm.at[wid], idx_v)
        pltpu.sync_copy(z_hbm.at[pl.ds(s * rps, rps)],
                        acc_sh.at[pl.ds(s * rps, rps)])
        plsc.subcore_barrier()
        for q in range(nq):
            lds[q].wait()
            if q + 1 < nq:
                lds[q + 1] = pltpu.async_copy(
                    tp_hbm.at[pl.ds(base + (q + 1) * qrows, qrows)],
                    rows_v.at[(q + 1) % 2], lsem)
            for jj in range(2):
                pltpu.sync_copy(rows_v.at[q % 2].at[pl.ds(jj * cw, cw)],
                                acc_sh.at[idx_v.at[2 * q + jj]], add=True)
        plsc.subcore_barrier()
        pltpu.sync_copy(acc_sh.at[pl.ds(s * rps, rps)],
                        out_hbm.at[c].at[pl.ds(s * rps, rps)])

    return k(tp64, src3, zinit)


def _combine_body(p_ref, na_ref, o_ref):
    ps = p_ref[0] + p_ref[1]
    cnt = jnp.maximum(ps[:, _D_NODE:_D_NODE + 1], 1.0)
    o_ref[...] = ps[:, :_D_OUT] / cnt + na_ref[...]


def _combine(partials, node_attr, interpret=False):
    n, d = node_attr.shape
    nt = 1024
    return pl.pallas_call(
        _combine_body,
        grid=(n // nt,),
        in_specs=[
            pl.BlockSpec((_NC, nt, _D_ROW), lambda i: (0, i, 0)),
            pl.BlockSpec((nt, d), lambda i: (i, 0)),
        ],
        out_specs=pl.BlockSpec((nt, d), lambda i: (i, 0)),
        out_shape=jax.ShapeDtypeStruct((n, d), jnp.float32),
        interpret=interpret,
    )(partials, node_attr)


def kernel(node_attr, edge_index, edge_attr, edge_sh, W1, b1, W2, b2):
    e = edge_index.shape[1]
    nchunk = e // (_NW * 128)
    src3 = edge_index[0].reshape(_NW, nchunk, 128)
    dst3 = edge_index[1].reshape(_NW, nchunk, 128)
    W2P, S64 = _prep_weights(W2, b2)
    node128 = jnp.pad(node_attr, ((0, 0), (0, 128 - node_attr.shape[1])))
    x = _sc_gather(node128, dst3)
    tp64 = _edge_compute(edge_attr, x, edge_sh, W1, b1, W2P, S64)
    n = node_attr.shape[0]
    zinit = jnp.zeros((n, _D_ROW), jnp.float32)
    partials = _sc_scatter(tp64, src3, zinit)
    return _combine(partials, node_attr)


# speedup vs baseline: 4.4850x; 1.0557x over previous
import functools

import jax
import jax.numpy as jnp
from jax import lax
from jax.experimental import pallas as pl
from jax.experimental.pallas import tpu as pltpu
from jax.experimental.pallas import tpu_sc as plsc

_NC = 2
_NS = 16
_NW = _NC * _NS

_D_NODE = 48
_D_OUT = 48
_D_PAD = 64
_D_ROW = 128
_INV_SQRT_FAN = 1.0 / (48.0 ** 0.5)

_E_TILE = 2048


def _edge_body(a_ref, x_ref, sh_ref, w1_ref, b1_ref, w2p_ref, s_ref, tp_ref):
    et = a_ref.shape[1]
    h = jax.lax.dot_general(a_ref[...].astype(jnp.bfloat16), w1_ref[...],
                            (((0,), (0,)), ((), ())),
                            preferred_element_type=jnp.float32)
    h = jnp.maximum(h + b1_ref[...], 0.0)
    sc = jnp.transpose(sh_ref[...]) * _INV_SQRT_FAN
    hp = jnp.concatenate([h * sc, sc], axis=1).astype(jnp.bfloat16)
    xr = x_ref[...][:, :_D_PAD]
    lane64 = jax.lax.broadcasted_iota(jnp.int32, xr.shape, 1)
    xp = jnp.where(lane64 < _D_NODE, xr, 0.0).astype(jnp.bfloat16)
    xt4 = jnp.tile(xp, (1, 4))
    acc = jnp.zeros((et, _D_ROW), jnp.float32)
    nchunks = w2p_ref.shape[1] // 256
    for j in range(nchunks):
        wr_j = jnp.dot(hp, w2p_ref[:, j * 256:(j + 1) * 256],
                       preferred_element_type=jnp.float32).astype(jnp.bfloat16)
        acc = acc + jnp.dot(wr_j * xt4, s_ref[j * 256:(j + 1) * 256, :],
                            preferred_element_type=jnp.float32)
    lane = jax.lax.broadcasted_iota(jnp.int32, acc.shape, 1)
    tp_ref[...] = jnp.where(lane == _D_NODE, 1.0, acc)


def _edge_compute(edge_attr, x, edge_sh, W1, b1, W2P, S64, interpret=False):
    E, d_edge = edge_attr.shape
    kw = _D_OUT * _D_PAD
    grid = (E // _E_TILE,)
    return pl.pallas_call(
        _edge_body,
        grid=grid,
        in_specs=[
            pl.BlockSpec((d_edge, _E_TILE), lambda i: (0, i)),
            pl.BlockSpec((_E_TILE, 128), lambda i: (i, 0)),
            pl.BlockSpec((1, _E_TILE), lambda i: (0, i)),
            pl.BlockSpec((d_edge, d_edge), lambda i: (0, 0)),
            pl.BlockSpec((1, d_edge), lambda i: (0, 0)),
            pl.BlockSpec((d_edge + 1, kw), lambda i: (0, 0)),
            pl.BlockSpec((kw, _D_ROW), lambda i: (0, 0)),
        ],
        out_specs=pl.BlockSpec((_E_TILE, _D_ROW), lambda i: (i, 0)),
        out_shape=jax.ShapeDtypeStruct((E, _D_ROW), jnp.float32),
        interpret=interpret,
    )(edge_attr.T, x, edge_sh.reshape(1, E),
      W1.astype(jnp.bfloat16), b1.reshape(1, -1), W2P, S64)


def _prep_weights(W2, b2):
    d_hid = W2.shape[0]
    w2t = W2.reshape(d_hid, _D_NODE, _D_OUT).transpose(0, 2, 1)
    b2t = b2.reshape(_D_NODE, _D_OUT).T[None]
    w2p = jnp.concatenate([w2t, b2t], axis=0)
    w2p = jnp.pad(w2p, ((0, 0), (0, 0), (0, _D_PAD - _D_NODE)))
    W2P = w2p.reshape(d_hid + 1, _D_OUT * _D_PAD).astype(jnp.bfloat16)
    j = jnp.arange(_D_OUT * _D_PAD)
    S64 = ((j[:, None] // _D_PAD == jnp.arange(_D_ROW)[None, :])
           & (j[:, None] % _D_PAD < _D_NODE)).astype(jnp.bfloat16)
    return W2P, S64


def _sc_gather(node128, dst3):
    n, d = node128.shape
    nw, nchunk, cw = dst3.shape
    epw = nchunk * cw
    nbuf = 6
    mesh = plsc.VectorSubcoreMesh(core_axis_name="c", subcore_axis_name="s")

    @functools.partial(
        pl.kernel, mesh=mesh,
        out_type=jax.ShapeDtypeStruct((nw * epw, d), jnp.float32),
        scratch_types=[
            pltpu.VMEM((nchunk, cw), jnp.int32),
            pltpu.VMEM((nbuf * cw, d), jnp.float32),
            pltpu.SemaphoreType.DMA,
            pltpu.SemaphoreType.DMA,
        ],
    )
    def g(node_hbm, idx_hbm, x_hbm, idx_v, rows_v, gsem, wsem):
        wid = lax.axis_index("c") * _NS + lax.axis_index("s")
        base = wid * epw
        pltpu.sync_copy(idx_hbm.at[wid], idx_v)
        gs = {}
        for j in range(nbuf):
            gs[j] = pltpu.async_copy(node_hbm.at[idx_v.at[j]],
                                     rows_v.at[pl.ds(j * cw, cw)], gsem)
        ws = {}
        for j in range(nchunk):
            b = j % nbuf
            gs[j].wait()
            ws[j] = pltpu.async_copy(rows_v.at[pl.ds(b * cw, cw)],
                                     x_hbm.at[pl.ds(base + j * cw, cw)], wsem)
            nj = j + nbuf
            if nj < nchunk:
                ws[j].wait()
                gs[nj] = pltpu.async_copy(node_hbm.at[idx_v.at[nj]],
                                          rows_v.at[pl.ds(b * cw, cw)], gsem)
        for j in range(nchunk - nbuf, nchunk):
            ws[j].wait()

    return g(node128, dst3)


def _sc_scatter(tp64, src3, zinit):
    e, d = tp64.shape
    nw, nchunk, cw = src3.shape
    epw = nchunk * cw
    n = zinit.shape[0]
    rps = n // _NS
    qrows = 2 * cw
    nq = epw // qrows
    mesh = plsc.VectorSubcoreMesh(core_axis_name="c", subcore_axis_name="s")

    @functools.partial(
        pl.kernel, mesh=mesh,
        out_type=jax.ShapeDtypeStruct((_NC, n, d), jnp.float32),
        scratch_types=[
            pltpu.VMEM((nchunk, cw), jnp.int32),
            pltpu.VMEM((2, qrows, d), jnp.float32),
            pltpu.VMEM_SHARED((n, d), jnp.float32),
            pltpu.SemaphoreType.DMA,
        ],
    )
    def k(tp_hbm, idx_hbm, z_hbm, out_hbm, idx_v, rows_v, acc_sh, lsem):
        c = lax.axis_index("c")
        s = lax.axis_index("s")
        wid = c * _NS + s
        base = wid * epw
        lds = {0: pltpu.async_copy(tp_hbm.at[pl.ds(base, qrows)],
                                   rows_v.at[0], lsem)}
        pltpu.sync_copy(idx_hbm.at[wid], idx_v)
        pltpu.sync_copy(z_hbm.at[pl.ds(s * rps, rps)],
                        acc_sh.at[pl.ds(s * rps, rps)])
        plsc.subcore_barrier()
        for q in range(nq):
            lds[q].wait()
            if q + 1 < nq:
                lds[q + 1] = pltpu.async_copy(
                    tp_hbm.at[pl.ds(base + (q + 1) * qrows, qrows)],
                    rows_v.at[(q + 1) % 2], lsem)
            for jj in range(2):
                pltpu.sync_copy(rows_v.at[q % 2].at[pl.ds(jj * cw, cw)],
                                acc_sh.at[idx_v.at[2 * q + jj]], add=True)
        plsc.subcore_barrier()
        pltpu.sync_copy(acc_sh.at[pl.ds(s * rps, rps)],
                        out_hbm.at[c].at[pl.ds(s * rps, rps)])

    return k(tp64, src3, zinit)


def _combine_body(p_ref, na_ref, o_ref):
    ps = p_ref[0] + p_ref[1]
    cnt = jnp.maximum(ps[:, _D_NODE:_D_NODE + 1], 1.0)
    o_ref[...] = ps[:, :_D_OUT] / cnt + na_ref[...]


def _combine(partials, node_attr, interpret=False):
    n, d = node_attr.shape
    nt = 1024
    return pl.pallas_call(
        _combine_body,
        grid=(n // nt,),
        in_specs=[
            pl.BlockSpec((_NC, nt, _D_ROW), lambda i: (0, i, 0)),
            pl.BlockSpec((nt, d), lambda i: (i, 0)),
        ],
        out_specs=pl.BlockSpec((nt, d), lambda i: (i, 0)),
        out_shape=jax.ShapeDtypeStruct((n, d), jnp.float32),
        interpret=interpret,
    )(partials, node_attr)


def kernel(node_attr, edge_index, edge_attr, edge_sh, W1, b1, W2, b2):
    e = edge_index.shape[1]
    nchunk = e // (_NW * 128)
    src3 = edge_index[0].reshape(_NW, nchunk, 128)
    dst3 = edge_index[1].reshape(_NW, nchunk, 128)
    W2P, S64 = _prep_weights(W2, b2)
    node128 = jnp.pad(node_attr, ((0, 0), (0, 128 - node_attr.shape[1])))
    x = _sc_gather(node128, dst3)
    tp64 = _edge_compute(edge_attr, x, edge_sh, W1, b1, W2P, S64)
    n = node_attr.shape[0]
    zinit = jnp.zeros((n, _D_ROW), jnp.float32)
    partials = _sc_scatter(tp64, src3, zinit)
    return _combine(partials, node_attr)
